# baseline (speedup 1.0000x reference)
import functools

import jax
import jax.numpy as jnp
from jax import lax
from jax.experimental import pallas as pl
from jax.experimental.pallas import tpu as pltpu
from jax.experimental.pallas import tpu_sc as plsc

DIM = 16
NC = 2
NS = 16
NBT = 8
SPW = 100
S = 200
BT = 128


def _emb5(x5, table):
    mesh = plsc.VectorSubcoreMesh(core_axis_name="c", subcore_axis_name="s")

    @functools.partial(
        pl.kernel,
        mesh=mesh,
        compiler_params=pltpu.CompilerParams(use_tc_tiling_on_sc=False),
        out_type=jax.ShapeDtypeStruct((S, 2, BT, 8, 128), jnp.float32),
        scratch_types=[
            pltpu.VMEM((NBT, 128), jnp.int32),
            pltpu.VMEM((NBT, 128), jnp.int32),
            pltpu.VMEM((NBT, 128, DIM), jnp.float32),
            pltpu.VMEM((NBT, 128, DIM), jnp.float32),
            pltpu.SemaphoreType.DMA,
            pltpu.SemaphoreType.DMA,
            pltpu.SemaphoreType.DMA,
            pltpu.SemaphoreType.DMA,
            pltpu.SemaphoreType.DMA,
            pltpu.SemaphoreType.DMA,
        ],
    )
    def emb(x5_hbm, table_hbm, out_hbm, i0, i1, g0, g1,
            si0, si1, sg0, sg1, so0, so1):
        wid = lax.axis_index("s") * NC + lax.axis_index("c")
        bt0 = (wid % 16) * NBT
        s0 = (wid // 16) * SPW
        idx_v = (i0, i1)
        g_v = (g0, g1)
        s_i = (si0, si1)
        s_g = (sg0, sg1)
        s_o = (so0, so1)

        def idx_cp(s, b):
            return pltpu.make_async_copy(
                x5_hbm.at[s // 8, pl.ds(bt0, NBT), s % 8], idx_v[b], s_i[b])

        def gat_start(b):
            for bt in range(NBT):
                pltpu.make_async_copy(
                    table_hbm.at[idx_v[b].at[bt]], g_v[b].at[bt], s_g[b]).start()

        def gat_wait(b):
            for bt in range(NBT):
                pltpu.make_async_copy(
                    table_hbm.at[idx_v[b].at[bt]], g_v[b].at[bt], s_g[b]).wait()

        def out_cp(s, b, e):
            return pltpu.make_async_copy(
                g_v[b].at[:, :, e],
                out_hbm.at[s, e // 8, pl.ds(bt0, NBT), e % 8], s_o[b])

        def out_start(s, b):
            for e in range(DIM):
                out_cp(s, b, e).start()

        def out_wait(s, b):
            for e in range(DIM):
                out_cp(s, b, e).wait()

        idx_cp(s0, 0).start()
        idx_cp(s0, 0).wait()
        gat_start(0)
        idx_cp(s0 + 1, 1).start()

        def body(i, carry):
            s = s0 + i
            b = lax.rem(i, 2)

            @pl.when(b == 0)
            def _():
                gat_wait(0)
                out_start(s, 0)
                @pl.when(i + 2 < SPW)
                def _():
                    idx_cp(s + 2, 0).start()
                @pl.when(i + 1 < SPW)
                def _():
                    idx_cp(s + 1, 1).wait()
                    @pl.when(i >= 1)
                    def _():
                        out_wait(s - 1, 1)
                    gat_start(1)

            @pl.when(b == 1)
            def _():
                gat_wait(1)
                out_start(s, 1)
                @pl.when(i + 2 < SPW)
                def _():
                    idx_cp(s + 2, 1).start()
                @pl.when(i + 1 < SPW)
                def _():
                    idx_cp(s + 1, 0).wait()
                    out_wait(s - 1, 0)
                    gat_start(0)
            return carry

        lax.fori_loop(0, SPW, body, 0)
        out_wait(s0 + SPW - 2, 0)
        out_wait(s0 + SPW - 1, 1)

    return emb(x5, table)


def kernel(x, table):
    x5 = x.T.reshape(25, 8, 128, 128).transpose(0, 2, 1, 3)
    o5 = _emb5(x5, table)
    return o5.transpose(2, 4, 0, 1, 3).reshape(16384, 200, 16)


# speedup vs baseline: 60.9491x; 60.9491x over previous
import functools

import jax
import jax.numpy as jnp
from jax import lax
from jax.experimental import pallas as pl
from jax.experimental.pallas import tpu as pltpu
from jax.experimental.pallas import tpu_sc as plsc

DIM = 16
NC = 2
NS = 16
NBT = 8
SPW = 100
S = 200
B = 16384
CH = NBT * 128


def _gather_lin(x5, table):
    mesh = plsc.VectorSubcoreMesh(core_axis_name="c", subcore_axis_name="s")

    @functools.partial(
        pl.kernel,
        mesh=mesh,
        compiler_params=pltpu.CompilerParams(use_tc_tiling_on_sc=False),
        out_type=jax.ShapeDtypeStruct((S * B, DIM), jnp.float32),
        scratch_types=[
            pltpu.VMEM((NBT, 128), jnp.int32),
            pltpu.VMEM((NBT, 128), jnp.int32),
            pltpu.VMEM((CH, DIM), jnp.float32),
            pltpu.VMEM((CH, DIM), jnp.float32),
            pltpu.SemaphoreType.DMA,
            pltpu.SemaphoreType.DMA,
            pltpu.SemaphoreType.DMA,
            pltpu.SemaphoreType.DMA,
            pltpu.SemaphoreType.DMA,
            pltpu.SemaphoreType.DMA,
        ],
    )
    def emb(x5_hbm, table_hbm, out_hbm, i0, i1, g0, g1,
            si0, si1, sg0, sg1, so0, so1):
        wid = lax.axis_index("s") * NC + lax.axis_index("c")
        bt0 = (wid % 16) * NBT
        s0 = (wid // 16) * SPW
        idx_v = (i0, i1)
        g_v = (g0, g1)
        s_i = (si0, si1)
        s_g = (sg0, sg1)
        s_o = (so0, so1)

        def idx_cp(s, b):
            return pltpu.make_async_copy(
                x5_hbm.at[s // 8, pl.ds(bt0, NBT), s % 8], idx_v[b], s_i[b])

        def gat_start(b):
            for bt in range(NBT):
                pltpu.make_async_copy(
                    table_hbm.at[idx_v[b].at[bt]],
                    g_v[b].at[pl.ds(bt * 128, 128)], s_g[b]).start()

        def gat_wait(b):
            for bt in range(NBT):
                pltpu.make_async_copy(
                    table_hbm.at[idx_v[b].at[bt]],
                    g_v[b].at[pl.ds(bt * 128, 128)], s_g[b]).wait()

        def out_cp(s, b):
            return pltpu.make_async_copy(
                g_v[b], out_hbm.at[pl.ds(s * B + bt0 * 128, CH)], s_o[b])

        idx_cp(s0, 0).start()
        idx_cp(s0, 0).wait()
        gat_start(0)
        idx_cp(s0 + 1, 1).start()

        def body(i, carry):
            s = s0 + i
            b = lax.rem(i, 2)

            @pl.when(b == 0)
            def _():
                gat_wait(0)
                out_cp(s, 0).start()
                @pl.when(i + 2 < SPW)
                def _():
                    idx_cp(s + 2, 0).start()
                @pl.when(i + 1 < SPW)
                def _():
                    idx_cp(s + 1, 1).wait()
                    @pl.when(i >= 1)
                    def _():
                        out_cp(s - 1, 1).wait()
                    gat_start(1)

            @pl.when(b == 1)
            def _():
                gat_wait(1)
                out_cp(s, 1).start()
                @pl.when(i + 2 < SPW)
                def _():
                    idx_cp(s + 2, 1).start()
                @pl.when(i + 1 < SPW)
                def _():
                    idx_cp(s + 1, 0).wait()
                    out_cp(s - 1, 0).wait()
                    gat_start(0)
            return carry

        lax.fori_loop(0, SPW, body, 0)
        out_cp(s0 + SPW - 2, 0).wait()
        out_cp(s0 + SPW - 1, 1).wait()

    return emb(x5, table)


def kernel(x, table):
    x5 = x.T.reshape(25, 8, 128, 128).transpose(0, 2, 1, 3)
    g = _gather_lin(x5, table)
    return g.reshape(S, B, DIM).transpose(1, 0, 2)


# speedup vs baseline: 122.2193x; 2.0053x over previous
import functools

import jax
import jax.numpy as jnp
from jax import lax
from jax.experimental import pallas as pl
from jax.experimental.pallas import tpu as pltpu
from jax.experimental.pallas import tpu_sc as plsc

DIM = 16
NC = 2
NS = 16
SPW = 100
S = 200
B = 16384
CH = 1024


def _tc_xpose(x2):
    def body(in_ref, out_ref):
        a = in_ref[...]
        a3 = a.reshape(128, 8, 128)
        for s_lo in range(8):
            out_ref[pl.ds(s_lo * 128, 128), :] = a3[:, s_lo, :].T

    return pl.pallas_call(
        body,
        grid=(S // 8,),
        in_specs=[pl.BlockSpec((1024, 128), lambda i: (i, 0))],
        out_specs=pl.BlockSpec((1024, 128), lambda i: (i, 0)),
        out_shape=jax.ShapeDtypeStruct((S * B // 128, 128), jnp.int32),
    )(x2)


def _gather_lin(xt, table):
    mesh = plsc.VectorSubcoreMesh(core_axis_name="c", subcore_axis_name="s")

    @functools.partial(
        pl.kernel,
        mesh=mesh,
        compiler_params=pltpu.CompilerParams(use_tc_tiling_on_sc=False),
        out_type=jax.ShapeDtypeStruct((S * B, DIM), jnp.float32),
        scratch_types=[
            pltpu.VMEM((CH,), jnp.int32),
            pltpu.VMEM((CH,), jnp.int32),
            pltpu.VMEM((CH, DIM), jnp.float32),
            pltpu.VMEM((CH, DIM), jnp.float32),
            pltpu.SemaphoreType.DMA,
            pltpu.SemaphoreType.DMA,
            pltpu.SemaphoreType.DMA,
            pltpu.SemaphoreType.DMA,
            pltpu.SemaphoreType.DMA,
            pltpu.SemaphoreType.DMA,
        ],
    )
    def emb(xt_hbm, table_hbm, out_hbm, i0, i1, g0, g1,
            si0, si1, sg0, sg1, so0, so1):
        wid = lax.axis_index("s") * NC + lax.axis_index("c")
        c0 = (wid % 16) * CH
        s0 = (wid // 16) * SPW
        idx_v = (i0, i1)
        g_v = (g0, g1)
        s_i = (si0, si1)
        s_g = (sg0, sg1)
        s_o = (so0, so1)

        def idx_cp(s, b):
            return pltpu.make_async_copy(
                xt_hbm.at[pl.ds(s * B + c0, CH)], idx_v[b], s_i[b])

        def gat_start(b):
            for bt in range(CH // 128):
                pltpu.make_async_copy(
                    table_hbm.at[idx_v[b].at[pl.ds(bt * 128, 128)]],
                    g_v[b].at[pl.ds(bt * 128, 128)], s_g[b]).start()

        def gat_wait(b):
            for bt in range(CH // 128):
                pltpu.make_async_copy(
                    table_hbm.at[idx_v[b].at[pl.ds(bt * 128, 128)]],
                    g_v[b].at[pl.ds(bt * 128, 128)], s_g[b]).wait()

        def out_cp(s, b):
            return pltpu.make_async_copy(
                g_v[b], out_hbm.at[pl.ds(s * B + c0, CH)], s_o[b])

        idx_cp(s0, 0).start()
        idx_cp(s0, 0).wait()
        gat_start(0)
        idx_cp(s0 + 1, 1).start()

        def body(i, carry):
            s = s0 + i
            b = lax.rem(i, 2)

            @pl.when(b == 0)
            def _():
                gat_wait(0)
                out_cp(s, 0).start()
                @pl.when(i + 2 < SPW)
                def _():
                    idx_cp(s + 2, 0).start()
                @pl.when(i + 1 < SPW)
                def _():
                    idx_cp(s + 1, 1).wait()
                    @pl.when(i >= 1)
                    def _():
                        out_cp(s - 1, 1).wait()
                    gat_start(1)

            @pl.when(b == 1)
            def _():
                gat_wait(1)
                out_cp(s, 1).start()
                @pl.when(i + 2 < SPW)
                def _():
                    idx_cp(s + 2, 1).start()
                @pl.when(i + 1 < SPW)
                def _():
                    idx_cp(s + 1, 0).wait()
                    out_cp(s - 1, 0).wait()
                    gat_start(0)
            return carry

        lax.fori_loop(0, SPW, body, 0)
        out_cp(s0 + SPW - 2, 0).wait()
        out_cp(s0 + SPW - 1, 1).wait()

    return emb(xt, table)


def _tc_shuffle(g2):
    def body(in_ref, out_ref):
        a = in_ref[...]
        a3 = a.reshape(128, 16, 128)
        for h in range(16):
            t = a3[:, h, :].T
            t4 = t.reshape(8, 2, 8, 128)
            w = t4.transpose(1, 0, 2, 3).reshape(2, 64, 128)
            out_ref[pl.ds(h * 64, 64), :] = w[0]
            out_ref[pl.ds(1024 + h * 64, 64), :] = w[1]

    return pl.pallas_call(
        body,
        grid=(S,),
        in_specs=[pl.BlockSpec((2048, 128), lambda s: (s, 0))],
        out_specs=pl.BlockSpec((2048, 128), lambda s: (s, 0)),
        out_shape=jax.ShapeDtypeStruct((S * B * DIM // 128, 128), jnp.float32),
    )(g2)


def kernel(x, table):
    x2 = x.T.reshape(25, 8, 128, 128).transpose(0, 2, 1, 3).reshape(
        S * B // 128, 128)
    xt = _tc_xpose(x2).reshape(S * B)
    g = _gather_lin(xt, table)
    o2 = _tc_shuffle(g.reshape(S * B * DIM // 128, 128))
    o5 = o2.reshape(S, 2, B // 128, 8, 128)
    return o5.transpose(2, 4, 0, 1, 3).reshape(B, S, DIM)


# speedup vs baseline: 127.7570x; 1.0453x over previous
import functools

import jax
import jax.numpy as jnp
from jax import lax
from jax.experimental import pallas as pl
from jax.experimental.pallas import tpu as pltpu
from jax.experimental.pallas import tpu_sc as plsc

DIM = 16
NC = 2
NS = 16
SPW = 50
S = 200
B = 16384
CH = 2048


def _tc_xpose(x2):
    def body(in_ref, out_ref):
        a = in_ref[...]
        a3 = a.reshape(128, 8, 128)
        for s_lo in range(8):
            out_ref[pl.ds(s_lo * 128, 128), :] = a3[:, s_lo, :].T

    return pl.pallas_call(
        body,
        grid=(S // 8,),
        in_specs=[pl.BlockSpec((1024, 128), lambda i: (i, 0))],
        out_specs=pl.BlockSpec((1024, 128), lambda i: (i, 0)),
        out_shape=jax.ShapeDtypeStruct((S * B // 128, 128), jnp.int32),
    )(x2)


def _gather_lin(xt, table):
    mesh = plsc.VectorSubcoreMesh(core_axis_name="c", subcore_axis_name="s")

    @functools.partial(
        pl.kernel,
        mesh=mesh,
        compiler_params=pltpu.CompilerParams(use_tc_tiling_on_sc=False),
        out_type=jax.ShapeDtypeStruct((S * B, DIM), jnp.float32),
        scratch_types=[
            pltpu.VMEM((CH,), jnp.int32),
            pltpu.VMEM((CH,), jnp.int32),
            pltpu.VMEM((CH, DIM), jnp.float32),
            pltpu.VMEM((CH, DIM), jnp.float32),
            pltpu.SemaphoreType.DMA,
            pltpu.SemaphoreType.DMA,
            pltpu.SemaphoreType.DMA,
            pltpu.SemaphoreType.DMA,
            pltpu.SemaphoreType.DMA,
            pltpu.SemaphoreType.DMA,
        ],
    )
    def emb(xt_hbm, table_hbm, out_hbm, i0, i1, g0, g1,
            si0, si1, sg0, sg1, so0, so1):
        wid = lax.axis_index("s") * NC + lax.axis_index("c")
        c0 = (wid % 8) * CH
        s0 = (wid // 8) * SPW
        idx_v = (i0, i1)
        g_v = (g0, g1)
        s_i = (si0, si1)
        s_g = (sg0, sg1)
        s_o = (so0, so1)

        def idx_cp(s, b):
            return pltpu.make_async_copy(
                xt_hbm.at[pl.ds(s * B + c0, CH)], idx_v[b], s_i[b])

        def gat_start(b):
            for bt in range(CH // 128):
                pltpu.make_async_copy(
                    table_hbm.at[idx_v[b].at[pl.ds(bt * 128, 128)]],
                    g_v[b].at[pl.ds(bt * 128, 128)], s_g[b]).start()

        def gat_wait(b):
            for bt in range(CH // 128):
                pltpu.make_async_copy(
                    table_hbm.at[idx_v[b].at[pl.ds(bt * 128, 128)]],
                    g_v[b].at[pl.ds(bt * 128, 128)], s_g[b]).wait()

        def out_cp(s, b):
            return pltpu.make_async_copy(
                g_v[b], out_hbm.at[pl.ds(s * B + c0, CH)], s_o[b])

        idx_cp(s0, 0).start()
        idx_cp(s0, 0).wait()
        gat_start(0)
        idx_cp(s0 + 1, 1).start()

        def body(i, carry):
            s = s0 + i
            b = lax.rem(i, 2)

            @pl.when(b == 0)
            def _():
                gat_wait(0)
                out_cp(s, 0).start()
                @pl.when(i + 2 < SPW)
                def _():
                    idx_cp(s + 2, 0).start()
                @pl.when(i + 1 < SPW)
                def _():
                    idx_cp(s + 1, 1).wait()
                    @pl.when(i >= 1)
                    def _():
                        out_cp(s - 1, 1).wait()
                    gat_start(1)

            @pl.when(b == 1)
            def _():
                gat_wait(1)
                out_cp(s, 1).start()
                @pl.when(i + 2 < SPW)
                def _():
                    idx_cp(s + 2, 1).start()
                @pl.when(i + 1 < SPW)
                def _():
                    idx_cp(s + 1, 0).wait()
                    out_cp(s - 1, 0).wait()
                    gat_start(0)
            return carry

        lax.fori_loop(0, SPW, body, 0)
        out_cp(s0 + SPW - 2, 0).wait()
        out_cp(s0 + SPW - 1, 1).wait()

    return emb(xt, table)


def _tc_shuffle(g2):
    def body(in_ref, out_ref):
        a = in_ref[...]
        a3 = a.reshape(128, 16, 128)
        for h in range(16):
            t = a3[:, h, :].T
            t4 = t.reshape(8, 2, 8, 128)
            w = t4.transpose(1, 0, 2, 3).reshape(2, 64, 128)
            out_ref[pl.ds(h * 64, 64), :] = w[0]
            out_ref[pl.ds(1024 + h * 64, 64), :] = w[1]

    return pl.pallas_call(
        body,
        grid=(S,),
        in_specs=[pl.BlockSpec((2048, 128), lambda s: (s, 0))],
        out_specs=pl.BlockSpec((2048, 128), lambda s: (s, 0)),
        out_shape=jax.ShapeDtypeStruct((S * B * DIM // 128, 128), jnp.float32),
    )(g2)


def kernel(x, table):
    x2 = x.T.reshape(25, 8, 128, 128).transpose(0, 2, 1, 3).reshape(
        S * B // 128, 128)
    xt = _tc_xpose(x2).reshape(S * B)
    g = _gather_lin(xt, table)
    o2 = _tc_shuffle(g.reshape(S * B * DIM // 128, 128))
    o5 = o2.reshape(S, 2, B // 128, 8, 128)
    return o5.transpose(2, 4, 0, 1, 3).reshape(B, S, DIM)


# speedup vs baseline: 136.2064x; 1.0661x over previous
import functools

import jax
import jax.numpy as jnp
from jax import lax
from jax.experimental import pallas as pl
from jax.experimental.pallas import tpu as pltpu
from jax.experimental.pallas import tpu_sc as plsc

DIM = 16
NC = 2
NS = 16
SPW = 25
S = 200
SH = 100
B = 16384
CH = 2048


def _tc_xpose(x2):
    def body(in_ref, out_ref):
        a = in_ref[...]
        a3 = a.reshape(128, 8, 128)
        for s_lo in range(8):
            out_ref[pl.ds(s_lo * 128, 128), :] = a3[:, s_lo, :].T

    return pl.pallas_call(
        body,
        grid=(S // 8,),
        in_specs=[pl.BlockSpec((1024, 128), lambda i: (i, 0))],
        out_specs=pl.BlockSpec((1024, 128), lambda i: (i, 0)),
        out_shape=jax.ShapeDtypeStruct((S * B // 128, 128), jnp.int32),
    )(x2)


def _gather_lin(xt, table, half):
    mesh = plsc.VectorSubcoreMesh(core_axis_name="c", subcore_axis_name="s")

    @functools.partial(
        pl.kernel,
        mesh=mesh,
        compiler_params=pltpu.CompilerParams(use_tc_tiling_on_sc=False),
        out_type=jax.ShapeDtypeStruct((SH * B, DIM), jnp.float32),
        scratch_types=[
            pltpu.VMEM((CH,), jnp.int32),
            pltpu.VMEM((CH,), jnp.int32),
            pltpu.VMEM((CH, DIM), jnp.float32),
            pltpu.VMEM((CH, DIM), jnp.float32),
            pltpu.SemaphoreType.DMA,
            pltpu.SemaphoreType.DMA,
            pltpu.SemaphoreType.DMA,
            pltpu.SemaphoreType.DMA,
            pltpu.SemaphoreType.DMA,
            pltpu.SemaphoreType.DMA,
        ],
    )
    def emb(xt_hbm, table_hbm, out_hbm, i0, i1, g0, g1,
            si0, si1, sg0, sg1, so0, so1):
        wid = lax.axis_index("s") * NC + lax.axis_index("c")
        c0 = (wid % 8) * CH
        s0 = (wid // 8) * SPW
        idx_v = (i0, i1)
        g_v = (g0, g1)
        s_i = (si0, si1)
        s_g = (sg0, sg1)
        s_o = (so0, so1)

        def idx_cp(s, b):
            return pltpu.make_async_copy(
                xt_hbm.at[pl.ds((half * SH + s) * B + c0, CH)],
                idx_v[b], s_i[b])

        def gat_start(b):
            for bt in range(CH // 128):
                pltpu.make_async_copy(
                    table_hbm.at[idx_v[b].at[pl.ds(bt * 128, 128)]],
                    g_v[b].at[pl.ds(bt * 128, 128)], s_g[b]).start()

        def gat_wait(b):
            for bt in range(CH // 128):
                pltpu.make_async_copy(
                    table_hbm.at[idx_v[b].at[pl.ds(bt * 128, 128)]],
                    g_v[b].at[pl.ds(bt * 128, 128)], s_g[b]).wait()

        def out_cp(s, b):
            return pltpu.make_async_copy(
                g_v[b], out_hbm.at[pl.ds(s * B + c0, CH)], s_o[b])

        idx_cp(s0, 0).start()
        idx_cp(s0, 0).wait()
        gat_start(0)
        idx_cp(s0 + 1, 1).start()

        def body(i, carry):
            s = s0 + i
            b = lax.rem(i, 2)

            @pl.when(b == 0)
            def _():
                gat_wait(0)
                out_cp(s, 0).start()
                @pl.when(i + 2 < SPW)
                def _():
                    idx_cp(s + 2, 0).start()
                @pl.when(i + 1 < SPW)
                def _():
                    idx_cp(s + 1, 1).wait()
                    @pl.when(i >= 1)
                    def _():
                        out_cp(s - 1, 1).wait()
                    gat_start(1)

            @pl.when(b == 1)
            def _():
                gat_wait(1)
                out_cp(s, 1).start()
                @pl.when(i + 2 < SPW)
                def _():
                    idx_cp(s + 2, 1).start()
                @pl.when(i + 1 < SPW)
                def _():
                    idx_cp(s + 1, 0).wait()
                    out_cp(s - 1, 0).wait()
                    gat_start(0)
            return carry

        lax.fori_loop(0, SPW, body, 0)
        out_cp(s0 + SPW - 2, 0).wait()
        out_cp(s0 + SPW - 1, 1).wait()

    return emb(xt, table)


def _tc_shuffle(g2, half, prev=None):
    def body(in_ref, *refs):
        out_ref = refs[-1]
        a = in_ref[...]
        a3 = a.reshape(128, 16, 128)
        for h in range(16):
            t = a3[:, h, :].T
            t4 = t.reshape(8, 2, 8, 128)
            w = t4.transpose(1, 0, 2, 3).reshape(2, 64, 128)
            out_ref[pl.ds(h * 64, 64), :] = w[0]
            out_ref[pl.ds(1024 + h * 64, 64), :] = w[1]

    in_specs = [pl.BlockSpec((2048, 128), lambda s: (s, 0))]
    args = (g2,)
    aliases = {}
    if prev is not None:
        in_specs.append(pl.BlockSpec(memory_space=pl.ANY))
        args = (g2, prev)
        aliases = {1: 0}
    return pl.pallas_call(
        body,
        grid=(SH,),
        in_specs=in_specs,
        out_specs=pl.BlockSpec((2048, 128), lambda s, _h=half: (s + _h * SH, 0)),
        out_shape=jax.ShapeDtypeStruct((S * B * DIM // 128, 128), jnp.float32),
        input_output_aliases=aliases,
    )(*args)


def kernel(x, table):
    x2 = x.T.reshape(25, 8, 128, 128).transpose(0, 2, 1, 3).reshape(
        S * B // 128, 128)
    xt = _tc_xpose(x2).reshape(S * B)
    g0 = _gather_lin(xt, table, 0)
    g1 = _gather_lin(xt, table, 1)
    oa = _tc_shuffle(g0.reshape(SH * B * DIM // 128, 128), 0)
    o2 = _tc_shuffle(g1.reshape(SH * B * DIM // 128, 128), 1, oa)
    o5 = o2.reshape(S, 2, B // 128, 8, 128)
    return o5.transpose(2, 4, 0, 1, 3).reshape(B, S, DIM)


# speedup vs baseline: 140.7902x; 1.0337x over previous
import functools

import jax
import jax.numpy as jnp
from jax import lax
from jax.experimental import pallas as pl
from jax.experimental.pallas import tpu as pltpu
from jax.experimental.pallas import tpu_sc as plsc

DIM = 16
NC = 2
NS = 16
SPW = 25
S = 200
SH = 50
B = 16384
CH = 1024


def _tc_xpose(x2):
    def body(in_ref, out_ref):
        a = in_ref[...]
        a3 = a.reshape(128, 8, 128)
        for s_lo in range(8):
            out_ref[pl.ds(s_lo * 128, 128), :] = a3[:, s_lo, :].T

    return pl.pallas_call(
        body,
        grid=(S // 8,),
        in_specs=[pl.BlockSpec((1024, 128), lambda i: (i, 0))],
        out_specs=pl.BlockSpec((1024, 128), lambda i: (i, 0)),
        out_shape=jax.ShapeDtypeStruct((S * B // 128, 128), jnp.int32),
    )(x2)


def _gather_lin(xt, table, half):
    mesh = plsc.VectorSubcoreMesh(core_axis_name="c", subcore_axis_name="s")

    @functools.partial(
        pl.kernel,
        mesh=mesh,
        compiler_params=pltpu.CompilerParams(use_tc_tiling_on_sc=False),
        out_type=jax.ShapeDtypeStruct((SH * B, DIM), jnp.float32),
        scratch_types=[
            pltpu.VMEM((CH,), jnp.int32),
            pltpu.VMEM((CH,), jnp.int32),
            pltpu.VMEM((CH, DIM), jnp.float32),
            pltpu.VMEM((CH, DIM), jnp.float32),
            pltpu.SemaphoreType.DMA,
            pltpu.SemaphoreType.DMA,
            pltpu.SemaphoreType.DMA,
            pltpu.SemaphoreType.DMA,
            pltpu.SemaphoreType.DMA,
            pltpu.SemaphoreType.DMA,
        ],
    )
    def emb(xt_hbm, table_hbm, out_hbm, i0, i1, g0, g1,
            si0, si1, sg0, sg1, so0, so1):
        wid = lax.axis_index("s") * NC + lax.axis_index("c")
        c0 = (wid % 16) * CH
        s0 = (wid // 16) * SPW
        idx_v = (i0, i1)
        g_v = (g0, g1)
        s_i = (si0, si1)
        s_g = (sg0, sg1)
        s_o = (so0, so1)

        def idx_cp(s, b):
            return pltpu.make_async_copy(
                xt_hbm.at[pl.ds((half * SH + s) * B + c0, CH)],
                idx_v[b], s_i[b])

        def gat_start(b):
            for bt in range(CH // 128):
                pltpu.make_async_copy(
                    table_hbm.at[idx_v[b].at[pl.ds(bt * 128, 128)]],
                    g_v[b].at[pl.ds(bt * 128, 128)], s_g[b]).start()

        def gat_wait(b):
            for bt in range(CH // 128):
                pltpu.make_async_copy(
                    table_hbm.at[idx_v[b].at[pl.ds(bt * 128, 128)]],
                    g_v[b].at[pl.ds(bt * 128, 128)], s_g[b]).wait()

        def out_cp(s, b):
            return pltpu.make_async_copy(
                g_v[b], out_hbm.at[pl.ds(s * B + c0, CH)], s_o[b])

        idx_cp(s0, 0).start()
        idx_cp(s0, 0).wait()
        gat_start(0)
        idx_cp(s0 + 1, 1).start()

        def body(i, carry):
            s = s0 + i
            b = lax.rem(i, 2)

            @pl.when(b == 0)
            def _():
                gat_wait(0)
                out_cp(s, 0).start()
                @pl.when(i + 2 < SPW)
                def _():
                    idx_cp(s + 2, 0).start()
                @pl.when(i + 1 < SPW)
                def _():
                    idx_cp(s + 1, 1).wait()
                    @pl.when(i >= 1)
                    def _():
                        out_cp(s - 1, 1).wait()
                    gat_start(1)

            @pl.when(b == 1)
            def _():
                gat_wait(1)
                out_cp(s, 1).start()
                @pl.when(i + 2 < SPW)
                def _():
                    idx_cp(s + 2, 1).start()
                @pl.when(i + 1 < SPW)
                def _():
                    idx_cp(s + 1, 0).wait()
                    out_cp(s - 1, 0).wait()
                    gat_start(0)
            return carry

        lax.fori_loop(0, SPW, body, 0)
        out_cp(s0 + SPW - 2, 0).wait()
        out_cp(s0 + SPW - 1, 1).wait()

    return emb(xt, table)


def _tc_shuffle(g2, half, prev=None):
    def body(in_ref, *refs):
        out_ref = refs[-1]
        a = in_ref[...]
        a3 = a.reshape(128, 16, 128)
        for h in range(16):
            t = a3[:, h, :].T
            t4 = t.reshape(8, 2, 8, 128)
            w = t4.transpose(1, 0, 2, 3).reshape(2, 64, 128)
            out_ref[pl.ds(h * 64, 64), :] = w[0]
            out_ref[pl.ds(1024 + h * 64, 64), :] = w[1]

    in_specs = [pl.BlockSpec((2048, 128), lambda s: (s, 0))]
    args = (g2,)
    aliases = {}
    if prev is not None:
        in_specs.append(pl.BlockSpec(memory_space=pl.ANY))
        args = (g2, prev)
        aliases = {1: 0}
    return pl.pallas_call(
        body,
        grid=(SH,),
        in_specs=in_specs,
        out_specs=pl.BlockSpec((2048, 128), lambda s, _h=half: (s + _h * SH, 0)),
        out_shape=jax.ShapeDtypeStruct((S * B * DIM // 128, 128), jnp.float32),
        input_output_aliases=aliases,
    )(*args)


def kernel(x, table):
    x2 = x.T.reshape(25, 8, 128, 128).transpose(0, 2, 1, 3).reshape(
        S * B // 128, 128)
    xt = _tc_xpose(x2).reshape(S * B)
    o2 = None
    for q in range(S // SH):
        gq = _gather_lin(xt, table, q)
        o2 = _tc_shuffle(gq.reshape(SH * B * DIM // 128, 128), q, o2)
    o5 = o2.reshape(S, 2, B // 128, 8, 128)
    return o5.transpose(2, 4, 0, 1, 3).reshape(B, S, DIM)
